# Initial kernel scaffold; baseline (speedup 1.0000x reference)
import functools

import jax
import jax.numpy as jnp
from jax import lax
from jax.experimental import pallas as pl
from jax.experimental.pallas import tpu as pltpu
from jax.experimental.pallas import tpu_sc as plsc

B = 4096
S = 50
V = 100000
D = 128
T = B * S
L = 16
NC = 2
NS = 16
NW = NC * NS
TPW = T // NW
C = 128
NCHUNK = TPW // C

_mesh = plsc.VectorSubcoreMesh(core_axis_name="c", subcore_axis_name="s")


@functools.partial(
    pl.kernel,
    mesh=_mesh,
    out_type=jax.ShapeDtypeStruct((T, 2 * D), jnp.float32),
    scratch_types=[
        pltpu.VMEM((C,), jnp.int32),
        pltpu.VMEM((C,), jnp.float32),
        pltpu.VMEM((C, D), jnp.float32),
        pltpu.VMEM((C, D), jnp.float32),
        pltpu.SemaphoreType.DMA,
    ],
)
def _emb_lookup(idx_hbm, mask_hbm, glove_hbm, fast_hbm, out_hbm,
                idxb, mb, gb, fb, sem):
    wid = lax.axis_index("s") * NC + lax.axis_index("c")
    wbase = wid * TPW

    def chunk_body(c, carry):
        base = wbase + c * C
        pltpu.sync_copy(idx_hbm.at[pl.ds(base, C)], idxb)
        pltpu.sync_copy(mask_hbm.at[pl.ds(base, C)], mb)
        cg = pltpu.async_copy(glove_hbm.at[idxb], gb, sem)
        cf = pltpu.async_copy(fast_hbm.at[idxb], fb, sem)
        cg.wait()
        cf.wait()

        def row_body(i, rcarry):
            m = plsc.load_gather(mb, [jnp.full((L,), i, jnp.int32)])
            for k in range(D // L):
                sl = pl.ds(k * L, L)
                gb[i, sl] = gb[i, sl] * m
                fb[i, sl] = fb[i, sl] * m
            return rcarry

        lax.fori_loop(0, C, row_body, 0)

        wg = pltpu.async_copy(gb, out_hbm.at[pl.ds(base, C), pl.ds(0, D)], sem)
        wf = pltpu.async_copy(fb, out_hbm.at[pl.ds(base, C), pl.ds(D, D)], sem)
        wg.wait()
        wf.wait()
        return carry

    lax.fori_loop(0, NCHUNK, chunk_body, 0)


def kernel(inputs, mask, glove_table, fasttext_table):
    idx = inputs.reshape(T)
    m = mask.reshape(T)
    out = _emb_lookup(idx, m, glove_table, fasttext_table)
    return out.reshape(B, S, 2 * D)


# speedup vs baseline: 3.0492x; 3.0492x over previous
import functools

import jax
import jax.numpy as jnp
from jax import lax
from jax.experimental import pallas as pl
from jax.experimental.pallas import tpu as pltpu
from jax.experimental.pallas import tpu_sc as plsc

B = 4096
S = 50
V = 100000
D = 128
T = B * S
L = 16
NC = 2
NS = 16
NW = NC * NS
TPW = T // NW
C = 128
NCHUNK = TPW // C

_mesh = plsc.VectorSubcoreMesh(core_axis_name="c", subcore_axis_name="s")

_DNUMS = lax.GatherDimensionNumbers(
    offset_dims=(), collapsed_slice_dims=(0,), start_index_map=(0,))


@functools.partial(
    pl.kernel,
    mesh=_mesh,
    out_type=jax.ShapeDtypeStruct((T, 2 * D), jnp.float32),
    scratch_types=[
        pltpu.VMEM((C,), jnp.int32),
        pltpu.VMEM((C,), jnp.float32),
        pltpu.VMEM((C, D), jnp.float32),
        pltpu.VMEM((C, D), jnp.float32),
        pltpu.SemaphoreType.DMA,
    ],
)
def _emb_lookup(idx_hbm, mask_hbm, glove_hbm, fast_hbm, out_hbm,
                idxb, mb, gb, fb, sem):
    wid = lax.axis_index("s") * NC + lax.axis_index("c")
    wbase = wid * TPW

    def chunk_body(c, carry):
        base = wbase + c * C
        pltpu.sync_copy(idx_hbm.at[pl.ds(base, C)], idxb)
        pltpu.sync_copy(mask_hbm.at[pl.ds(base, C)], mb)
        cg = pltpu.async_copy(glove_hbm.at[idxb], gb, sem)
        cf = pltpu.async_copy(fast_hbm.at[idxb], fb, sem)
        cg.wait()
        cf.wait()

        def group_body(g, rcarry):
            mvec = mb[pl.ds(g * L, L)]
            for j in range(L):
                m = lax.gather(
                    mvec,
                    jnp.full((L, 1), j, jnp.int32),
                    _DNUMS,
                    (1,),
                    mode=lax.GatherScatterMode.PROMISE_IN_BOUNDS,
                )
                i = g * L + j
                for k in range(D // L):
                    sl = pl.ds(k * L, L)
                    gb[i, sl] = gb[i, sl] * m
                    fb[i, sl] = fb[i, sl] * m
            return rcarry

        lax.fori_loop(0, C // L, group_body, 0)

        wg = pltpu.async_copy(gb, out_hbm.at[pl.ds(base, C), pl.ds(0, D)], sem)
        wf = pltpu.async_copy(fb, out_hbm.at[pl.ds(base, C), pl.ds(D, D)], sem)
        wg.wait()
        wf.wait()
        return carry

    lax.fori_loop(0, NCHUNK, chunk_body, 0)


def kernel(inputs, mask, glove_table, fasttext_table):
    idx = inputs.reshape(T)
    m = mask.reshape(T)
    out = _emb_lookup(idx, m, glove_table, fasttext_table)
    return out.reshape(B, S, 2 * D)
